# Initial kernel scaffold; baseline (speedup 1.0000x reference)
import functools

import jax
import jax.numpy as jnp
from jax import lax
from jax.experimental import pallas as pl
from jax.experimental.pallas import tpu as pltpu
from jax.experimental.pallas import tpu_sc as plsc

N = 10000
E = 320000
D = 128
NW = 32
CHUNK = 128
CPW = 80
EPAD = NW * CPW * CHUNK
JUNK = N
NPAD = 10240
RPT = NPAD // 16
ZROWS = 64
BR = 1000

_mesh = plsc.VectorSubcoreMesh(
    core_axis_name="c", subcore_axis_name="s", num_cores=2, num_subcores=16)


def _sc_degree(dst3, ones_rows, zrows):
  out_t = jax.ShapeDtypeStruct((2, NPAD, 16), jnp.float32)

  @functools.partial(
      pl.kernel, out_type=out_t, mesh=_mesh,
      scratch_types=[
          pltpu.VMEM((CHUNK,), jnp.int32),
          pltpu.VMEM((CHUNK, 16), jnp.float32),
          pltpu.VMEM((ZROWS, 16), jnp.float32),
          pltpu.VMEM_SHARED((NPAD, 16), jnp.float32),
      ])
  def k(dst_hbm, ones_hbm, z_hbm, out_hbm, didx_v, ones_v, z_v, acc):
    cid = lax.axis_index("c")
    sid = lax.axis_index("s")
    wid = sid * 2 + cid
    pltpu.sync_copy(ones_hbm, ones_v)
    pltpu.sync_copy(z_hbm, z_v)
    base = sid * RPT
    for r in range(RPT // ZROWS):
      pltpu.sync_copy(z_v, acc.at[pl.ds(base + r * ZROWS, ZROWS)])
    plsc.subcore_barrier()

    @pl.loop(0, CPW)
    def _(c):
      pltpu.sync_copy(dst_hbm.at[wid, c], didx_v)
      pltpu.sync_copy(ones_v, acc.at[didx_v], add=True)

    plsc.subcore_barrier()
    pltpu.sync_copy(acc.at[pl.ds(base, RPT)],
                    out_hbm.at[cid, pl.ds(base, RPT)])

  return k(dst3, ones_rows, zrows)


def _sc_agg(tp, src3, dst3, zrows):
  out_t = jax.ShapeDtypeStruct((2, NPAD, D), jnp.float32)

  @functools.partial(
      pl.kernel, out_type=out_t, mesh=_mesh,
      scratch_types=[
          pltpu.VMEM((CHUNK,), jnp.int32),
          pltpu.VMEM((CHUNK,), jnp.int32),
          pltpu.VMEM((CHUNK, D), jnp.float32),
          pltpu.VMEM((ZROWS, D), jnp.float32),
          pltpu.VMEM_SHARED((NPAD, D), jnp.float32),
      ])
  def k(tp_hbm, src_hbm, dst_hbm, z_hbm, out_hbm,
        sidx_v, didx_v, rows_v, z_v, acc):
    cid = lax.axis_index("c")
    sid = lax.axis_index("s")
    wid = sid * 2 + cid
    pltpu.sync_copy(z_hbm, z_v)
    base = sid * RPT
    for r in range(RPT // ZROWS):
      pltpu.sync_copy(z_v, acc.at[pl.ds(base + r * ZROWS, ZROWS)])
    plsc.subcore_barrier()

    @pl.loop(0, CPW)
    def _(c):
      pltpu.sync_copy(src_hbm.at[wid, c], sidx_v)
      pltpu.sync_copy(dst_hbm.at[wid, c], didx_v)
      pltpu.sync_copy(tp_hbm.at[sidx_v], rows_v)
      pltpu.sync_copy(rows_v, acc.at[didx_v], add=True)

    plsc.subcore_barrier()
    pltpu.sync_copy(acc.at[pl.ds(base, RPT)],
                    out_hbm.at[cid, pl.ds(base, RPT)])

  return k(tp, src3, dst3, zrows)


def _tc_mm(x, w):
  def body(x_ref, w_ref, o_ref):
    o_ref[...] = jnp.dot(x_ref[...], w_ref[...],
                         preferred_element_type=jnp.float32)

  return pl.pallas_call(
      body,
      grid=(N // BR,),
      in_specs=[pl.BlockSpec((BR, D), lambda i: (i, 0)),
                pl.BlockSpec((D, D), lambda i: (0, 0))],
      out_specs=pl.BlockSpec((BR, D), lambda i: (i, 0)),
      out_shape=jax.ShapeDtypeStruct((N, D), jnp.float32),
  )(x, w)


def _tc_scale(degp, t):
  def body(dg_ref, t_ref, dv_ref, tp_ref):
    dsum = dg_ref[0, :, 0:1] + dg_ref[1, :, 0:1] + 1.0
    dv = lax.rsqrt(dsum)
    dv_ref[...] = dv
    tp_ref[...] = t_ref[...] * dv

  return pl.pallas_call(
      body,
      grid=(N // BR,),
      in_specs=[pl.BlockSpec((2, BR, 16), lambda i: (0, i, 0)),
                pl.BlockSpec((BR, D), lambda i: (i, 0))],
      out_specs=[pl.BlockSpec((BR, 1), lambda i: (i, 0)),
                 pl.BlockSpec((BR, D), lambda i: (i, 0))],
      out_shape=[jax.ShapeDtypeStruct((N, 1), jnp.float32),
                 jax.ShapeDtypeStruct((N, D), jnp.float32)],
  )(degp, t)


def _tc_layer(parts, tp, dinv, b2d, wn):
  def body(p_ref, t_ref, dv_ref, b_ref, w_ref, o_ref):
    dv = dv_ref[...]
    s = p_ref[0] + p_ref[1] + t_ref[...]
    h = jnp.maximum(dv * s + b_ref[...], 0.0)
    o_ref[...] = jnp.dot(h, w_ref[...],
                         preferred_element_type=jnp.float32) * dv

  return pl.pallas_call(
      body,
      grid=(N // BR,),
      in_specs=[pl.BlockSpec((2, BR, D), lambda i: (0, i, 0)),
                pl.BlockSpec((BR, D), lambda i: (i, 0)),
                pl.BlockSpec((BR, 1), lambda i: (i, 0)),
                pl.BlockSpec((1, D), lambda i: (0, 0)),
                pl.BlockSpec((D, D), lambda i: (0, 0))],
      out_specs=pl.BlockSpec((BR, D), lambda i: (i, 0)),
      out_shape=jax.ShapeDtypeStruct((N, D), jnp.float32),
  )(parts, tp, dinv, b2d, wn)


def _tc_final(parts, tp, dinv, b2d):
  def body(p_ref, t_ref, dv_ref, b_ref, o_ref):
    s = p_ref[0] + p_ref[1] + t_ref[...]
    o_ref[...] = dv_ref[...] * s + b_ref[...]

  return pl.pallas_call(
      body,
      grid=(N // BR,),
      in_specs=[pl.BlockSpec((2, BR, D), lambda i: (0, i, 0)),
                pl.BlockSpec((BR, D), lambda i: (i, 0)),
                pl.BlockSpec((BR, 1), lambda i: (i, 0)),
                pl.BlockSpec((1, D), lambda i: (0, 0))],
      out_specs=pl.BlockSpec((BR, D), lambda i: (i, 0)),
      out_shape=jax.ShapeDtypeStruct((N, D), jnp.float32),
  )(parts, tp, dinv, b2d)


def kernel(x, edge_index, batch, W1, b1, W2, b2, W3, b3):
  src = edge_index[0].astype(jnp.int32)
  dst = edge_index[1].astype(jnp.int32)
  src3 = jnp.concatenate(
      [src, jnp.zeros((EPAD - E,), jnp.int32)]).reshape(NW, CPW, CHUNK)
  dst3 = jnp.concatenate(
      [dst, jnp.full((EPAD - E,), JUNK, jnp.int32)]).reshape(NW, CPW, CHUNK)
  ones_rows = jnp.ones((CHUNK, 16), jnp.float32)
  z16 = jnp.zeros((ZROWS, 16), jnp.float32)
  zD = jnp.zeros((ZROWS, D), jnp.float32)
  b1r, b2r, b3r = b1.reshape(1, D), b2.reshape(1, D), b3.reshape(1, D)

  t1 = _tc_mm(x, W1)
  degp = _sc_degree(dst3, ones_rows, z16)
  dinv, t1p = _tc_scale(degp, t1)
  p1 = _sc_agg(t1p, src3, dst3, zD)
  t2p = _tc_layer(p1, t1p, dinv, b1r, W2)
  p2 = _sc_agg(t2p, src3, dst3, zD)
  t3p = _tc_layer(p2, t2p, dinv, b2r, W3)
  p3 = _sc_agg(t3p, src3, dst3, zD)
  return _tc_final(p3, t3p, dinv, b3r)


# speedup vs baseline: 6.5608x; 6.5608x over previous
import functools

import jax
import jax.numpy as jnp
from jax import lax
from jax.experimental import pallas as pl
from jax.experimental.pallas import tpu as pltpu
from jax.experimental.pallas import tpu_sc as plsc

N = 10000
E = 320000
D = 128
NW = 32
CHUNK = 128
CPW = 80
EPAD = NW * CPW * CHUNK
JUNK = N
NPAD = 10240
RPT = NPAD // 16
ZROWS = 64
BR = 1000

_mesh = plsc.VectorSubcoreMesh(
    core_axis_name="c", subcore_axis_name="s", num_cores=2, num_subcores=16)


def _sc_degree(dst_flat, ones_rows, zrows):
  out_t = jax.ShapeDtypeStruct((2 * NPAD, D), jnp.float32)

  @functools.partial(
      pl.kernel, out_type=out_t, mesh=_mesh,
      scratch_types=[
          pltpu.VMEM((CHUNK,), jnp.int32),
          pltpu.VMEM((CHUNK, D), jnp.float32),
          pltpu.VMEM((ZROWS, D), jnp.float32),
          pltpu.VMEM_SHARED((NPAD, D), jnp.float32),
      ])
  def k(dst_hbm, ones_hbm, z_hbm, out_hbm, didx_v, ones_v, z_v, acc):
    cid = lax.axis_index("c")
    sid = lax.axis_index("s")
    wid = sid * 2 + cid
    pltpu.sync_copy(ones_hbm, ones_v)
    pltpu.sync_copy(z_hbm, z_v)
    base = sid * RPT
    for r in range(RPT // ZROWS):
      pltpu.sync_copy(z_v, acc.at[pl.ds(base + r * ZROWS, ZROWS)])
    plsc.subcore_barrier()
    e0 = wid * CPW * CHUNK

    @pl.loop(0, CPW)
    def _(c):
      pltpu.sync_copy(dst_hbm.at[pl.ds(e0 + c * CHUNK, CHUNK)], didx_v)
      pltpu.sync_copy(ones_v, acc.at[didx_v], add=True)

    plsc.subcore_barrier()
    pltpu.sync_copy(acc.at[pl.ds(base, RPT)],
                    out_hbm.at[pl.ds(cid * NPAD + base, RPT)])

  return k(dst_flat, ones_rows, zrows)


def _sc_agg(tp, src_flat, dst_flat, zrows):
  out_t = jax.ShapeDtypeStruct((2 * NPAD, D), jnp.float32)

  @functools.partial(
      pl.kernel, out_type=out_t, mesh=_mesh,
      scratch_types=[
          pltpu.VMEM((CHUNK,), jnp.int32),
          pltpu.VMEM((CHUNK,), jnp.int32),
          pltpu.VMEM((CHUNK, D), jnp.float32),
          pltpu.VMEM((ZROWS, D), jnp.float32),
          pltpu.VMEM_SHARED((NPAD, D), jnp.float32),
      ])
  def k(tp_hbm, src_hbm, dst_hbm, z_hbm, out_hbm,
        sidx_v, didx_v, rows_v, z_v, acc):
    cid = lax.axis_index("c")
    sid = lax.axis_index("s")
    wid = sid * 2 + cid
    pltpu.sync_copy(z_hbm, z_v)
    base = sid * RPT
    for r in range(RPT // ZROWS):
      pltpu.sync_copy(z_v, acc.at[pl.ds(base + r * ZROWS, ZROWS)])
    plsc.subcore_barrier()
    e0 = wid * CPW * CHUNK

    @pl.loop(0, CPW)
    def _(c):
      pltpu.sync_copy(src_hbm.at[pl.ds(e0 + c * CHUNK, CHUNK)], sidx_v)
      pltpu.sync_copy(dst_hbm.at[pl.ds(e0 + c * CHUNK, CHUNK)], didx_v)
      pltpu.sync_copy(tp_hbm.at[sidx_v], rows_v)
      pltpu.sync_copy(rows_v, acc.at[didx_v], add=True)

    plsc.subcore_barrier()
    pltpu.sync_copy(acc.at[pl.ds(base, RPT)],
                    out_hbm.at[pl.ds(cid * NPAD + base, RPT)])

  return k(tp, src_flat, dst_flat, zrows)


def _tc_mm(x, w):
  def body(x_ref, w_ref, o_ref):
    o_ref[...] = jnp.dot(x_ref[...], w_ref[...],
                         preferred_element_type=jnp.float32)

  return pl.pallas_call(
      body,
      grid=(N // BR,),
      in_specs=[pl.BlockSpec((BR, D), lambda i: (i, 0)),
                pl.BlockSpec((D, D), lambda i: (0, 0))],
      out_specs=pl.BlockSpec((BR, D), lambda i: (i, 0)),
      out_shape=jax.ShapeDtypeStruct((N, D), jnp.float32),
  )(x, w)


def _tc_scale(degp, t):
  def body(dg_ref, t_ref, dv_ref, tp_ref):
    dsum = dg_ref[0, :, 0:1] + dg_ref[1, :, 0:1] + 1.0
    dv = lax.rsqrt(dsum)
    dv_ref[...] = dv
    tp_ref[...] = t_ref[...] * dv

  return pl.pallas_call(
      body,
      grid=(N // BR,),
      in_specs=[pl.BlockSpec((2, BR, D), lambda i: (0, i, 0)),
                pl.BlockSpec((BR, D), lambda i: (i, 0))],
      out_specs=[pl.BlockSpec((BR, 1), lambda i: (i, 0)),
                 pl.BlockSpec((BR, D), lambda i: (i, 0))],
      out_shape=[jax.ShapeDtypeStruct((N, 1), jnp.float32),
                 jax.ShapeDtypeStruct((N, D), jnp.float32)],
  )(degp, t)


def _tc_layer(parts, tp, dinv, b2d, wn):
  def body(p_ref, t_ref, dv_ref, b_ref, w_ref, o_ref):
    dv = dv_ref[...]
    s = p_ref[0] + p_ref[1] + t_ref[...]
    h = jnp.maximum(dv * s + b_ref[...], 0.0)
    o_ref[...] = jnp.dot(h, w_ref[...],
                         preferred_element_type=jnp.float32) * dv

  return pl.pallas_call(
      body,
      grid=(N // BR,),
      in_specs=[pl.BlockSpec((2, BR, D), lambda i: (0, i, 0)),
                pl.BlockSpec((BR, D), lambda i: (i, 0)),
                pl.BlockSpec((BR, 1), lambda i: (i, 0)),
                pl.BlockSpec((1, D), lambda i: (0, 0)),
                pl.BlockSpec((D, D), lambda i: (0, 0))],
      out_specs=pl.BlockSpec((BR, D), lambda i: (i, 0)),
      out_shape=jax.ShapeDtypeStruct((N, D), jnp.float32),
  )(parts, tp, dinv, b2d, wn)


def _tc_final(parts, tp, dinv, b2d):
  def body(p_ref, t_ref, dv_ref, b_ref, o_ref):
    s = p_ref[0] + p_ref[1] + t_ref[...]
    o_ref[...] = dv_ref[...] * s + b_ref[...]

  return pl.pallas_call(
      body,
      grid=(N // BR,),
      in_specs=[pl.BlockSpec((2, BR, D), lambda i: (0, i, 0)),
                pl.BlockSpec((BR, D), lambda i: (i, 0)),
                pl.BlockSpec((BR, 1), lambda i: (i, 0)),
                pl.BlockSpec((1, D), lambda i: (0, 0))],
      out_specs=pl.BlockSpec((BR, D), lambda i: (i, 0)),
      out_shape=jax.ShapeDtypeStruct((N, D), jnp.float32),
  )(parts, tp, dinv, b2d)


def kernel(x, edge_index, batch, W1, b1, W2, b2, W3, b3):
  src = edge_index[0].astype(jnp.int32)
  dst = edge_index[1].astype(jnp.int32)
  src_f = jnp.concatenate([src, jnp.zeros((EPAD - E,), jnp.int32)])
  dst_f = jnp.concatenate([dst, jnp.full((EPAD - E,), JUNK, jnp.int32)])
  ones_rows = jnp.ones((CHUNK, D), jnp.float32)
  zD = jnp.zeros((ZROWS, D), jnp.float32)
  b1r, b2r, b3r = b1.reshape(1, D), b2.reshape(1, D), b3.reshape(1, D)

  t1 = _tc_mm(x, W1)
  degp = _sc_degree(dst_f, ones_rows, zD).reshape(2, NPAD, D)
  dinv, t1p = _tc_scale(degp, t1)
  p1 = _sc_agg(t1p, src_f, dst_f, zD).reshape(2, NPAD, D)
  t2p = _tc_layer(p1, t1p, dinv, b1r, W2)
  p2 = _sc_agg(t2p, src_f, dst_f, zD).reshape(2, NPAD, D)
  t3p = _tc_layer(p2, t2p, dinv, b2r, W3)
  p3 = _sc_agg(t3p, src_f, dst_f, zD).reshape(2, NPAD, D)
  return _tc_final(p3, t3p, dinv, b3r)


# speedup vs baseline: 8.8328x; 1.3463x over previous
import functools

import jax
import jax.numpy as jnp
from jax import lax
from jax.experimental import pallas as pl
from jax.experimental.pallas import tpu as pltpu
from jax.experimental.pallas import tpu_sc as plsc

N = 10000
E = 320000
D = 128
NW = 32
CHUNK = 128
CPW = 80
EPAD = NW * CPW * CHUNK
JUNK = N
NPAD = 10240
RPT = NPAD // 16
ZROWS = 32
BR = 1000

_mesh = plsc.VectorSubcoreMesh(
    core_axis_name="c", subcore_axis_name="s", num_cores=2, num_subcores=16)


KBAT = 16


def _zero_acc(acc, z_hbm, z_v, base, zsem):
  pltpu.sync_copy(z_hbm, z_v)
  nz = RPT // ZROWS
  for r in range(nz):
    pltpu.async_copy(z_v, acc.at[pl.ds(base + r * ZROWS, ZROWS)], zsem)
  for r in range(nz):
    pltpu.make_async_copy(z_v, acc.at[pl.ds(base + r * ZROWS, ZROWS)],
                          zsem).wait()


def _sc_degree(dst2, ones_rows, zrows):
  out_t = jax.ShapeDtypeStruct((2 * NPAD, D), jnp.float32)

  @functools.partial(
      pl.kernel, out_type=out_t, mesh=_mesh,
      scratch_types=[
          pltpu.VMEM((CPW, CHUNK), jnp.int32),
          pltpu.VMEM((CHUNK, D), jnp.float32),
          pltpu.VMEM((ZROWS, D), jnp.float32),
          pltpu.VMEM_SHARED((NPAD, D), jnp.float32),
          pltpu.SemaphoreType.DMA,
          pltpu.SemaphoreType.DMA,
      ])
  def k(dst_hbm, ones_hbm, z_hbm, out_hbm, didx, ones_v, z_v, acc, sem, zsem):
    cid = lax.axis_index("c")
    sid = lax.axis_index("s")
    wid = sid * 2 + cid
    pltpu.sync_copy(dst_hbm.at[pl.ds(wid * CPW, CPW)], didx)
    pltpu.sync_copy(ones_hbm, ones_v)
    base = sid * RPT
    _zero_acc(acc, z_hbm, z_v, base, zsem)
    plsc.subcore_barrier()

    @pl.loop(0, CPW, step=KBAT)
    def _(c):
      for b in range(KBAT):
        pltpu.async_copy(ones_v, acc.at[didx.at[c + b]], sem, add=True)
      for b in range(KBAT):
        pltpu.make_async_copy(ones_v, acc.at[didx.at[c + b]], sem).wait()

    plsc.subcore_barrier()
    pltpu.sync_copy(acc.at[pl.ds(base, RPT)],
                    out_hbm.at[pl.ds(cid * NPAD + base, RPT)])

  return k(dst2, ones_rows, zrows)


def _sc_agg(tp, ei3, zrows):
  out_t = jax.ShapeDtypeStruct((2 * NPAD, D), jnp.float32)

  @functools.partial(
      pl.kernel, out_type=out_t, mesh=_mesh,
      scratch_types=[
          [pltpu.VMEM((2, CHUNK), jnp.int32)] * 4,
          [pltpu.VMEM((CHUNK, D), jnp.float32)] * 2,
          pltpu.VMEM((ZROWS, D), jnp.float32),
          pltpu.VMEM_SHARED((NPAD, D), jnp.float32),
          [pltpu.SemaphoreType.DMA] * 4,
          [pltpu.SemaphoreType.DMA] * 2,
          pltpu.SemaphoreType.DMA,
      ])
  def k(tp_hbm, ei_hbm, z_hbm, out_hbm, ibuf, rows, z_v, acc,
        isem, gsem, zsem):
    cid = lax.axis_index("c")
    sid = lax.axis_index("s")
    wid = sid * 2 + cid
    e0 = wid * CPW
    base = sid * RPT
    _zero_acc(acc, z_hbm, z_v, base, zsem)
    plsc.subcore_barrier()

    for b in range(2):
      pltpu.sync_copy(ei_hbm.at[e0 + b], ibuf[b])
      pltpu.async_copy(tp_hbm.at[ibuf[b].at[0]], rows[b], gsem[b])
    for b in range(2, 4):
      pltpu.async_copy(ei_hbm.at[e0 + b], ibuf[b], isem[b])

    def step(c, b4, more_idx):
      b2 = b4 % 2
      nb4 = (b4 + 2) % 4
      pltpu.make_async_copy(tp_hbm.at[ibuf[b2].at[0]], rows[b2],
                            gsem[b2]).wait()
      pltpu.sync_copy(rows[b2], acc.at[ibuf[b4].at[1]], add=True)
      if more_idx:
        pltpu.async_copy(ei_hbm.at[e0 + c + 4], ibuf[b4], isem[b4])
      pltpu.make_async_copy(ei_hbm.at[e0], ibuf[nb4], isem[nb4]).wait()
      pltpu.async_copy(tp_hbm.at[ibuf[nb4].at[0]], rows[b2], gsem[b2])

    @pl.loop(0, CPW - 4, step=4)
    def _(c):
      for b4 in range(4):
        step(c + b4, b4, more_idx=True)

    for cc in range(CPW - 4, CPW):
      b4 = cc % 4
      b2 = b4 % 2
      pltpu.make_async_copy(tp_hbm.at[ibuf[b2].at[0]], rows[b2],
                            gsem[b2]).wait()
      pltpu.sync_copy(rows[b2], acc.at[ibuf[b4].at[1]], add=True)
      if cc + 2 < CPW:
        nb4 = (b4 + 2) % 4
        pltpu.make_async_copy(ei_hbm.at[e0], ibuf[nb4], isem[nb4]).wait()
        pltpu.async_copy(tp_hbm.at[ibuf[nb4].at[0]], rows[b2], gsem[b2])

    plsc.subcore_barrier()
    pltpu.sync_copy(acc.at[pl.ds(base, RPT)],
                    out_hbm.at[pl.ds(cid * NPAD + base, RPT)])

  return k(tp, ei3, zrows)


def _tc_mm(x, w):
  def body(x_ref, w_ref, o_ref):
    o_ref[...] = jnp.dot(x_ref[...], w_ref[...],
                         preferred_element_type=jnp.float32)

  return pl.pallas_call(
      body,
      grid=(N // BR,),
      in_specs=[pl.BlockSpec((BR, D), lambda i: (i, 0)),
                pl.BlockSpec((D, D), lambda i: (0, 0))],
      out_specs=pl.BlockSpec((BR, D), lambda i: (i, 0)),
      out_shape=jax.ShapeDtypeStruct((N, D), jnp.float32),
  )(x, w)


def _tc_scale(degp, t):
  def body(dg_ref, t_ref, dv_ref, tp_ref):
    dsum = dg_ref[0, :, 0:1] + dg_ref[1, :, 0:1] + 1.0
    dv = lax.rsqrt(dsum)
    dv_ref[...] = dv
    tp_ref[...] = t_ref[...] * dv

  return pl.pallas_call(
      body,
      grid=(N // BR,),
      in_specs=[pl.BlockSpec((2, BR, D), lambda i: (0, i, 0)),
                pl.BlockSpec((BR, D), lambda i: (i, 0))],
      out_specs=[pl.BlockSpec((BR, 1), lambda i: (i, 0)),
                 pl.BlockSpec((BR, D), lambda i: (i, 0))],
      out_shape=[jax.ShapeDtypeStruct((N, 1), jnp.float32),
                 jax.ShapeDtypeStruct((N, D), jnp.float32)],
  )(degp, t)


def _tc_layer(parts, tp, dinv, b2d, wn):
  def body(p_ref, t_ref, dv_ref, b_ref, w_ref, o_ref):
    dv = dv_ref[...]
    s = p_ref[0] + p_ref[1] + t_ref[...]
    h = jnp.maximum(dv * s + b_ref[...], 0.0)
    o_ref[...] = jnp.dot(h, w_ref[...],
                         preferred_element_type=jnp.float32) * dv

  return pl.pallas_call(
      body,
      grid=(N // BR,),
      in_specs=[pl.BlockSpec((2, BR, D), lambda i: (0, i, 0)),
                pl.BlockSpec((BR, D), lambda i: (i, 0)),
                pl.BlockSpec((BR, 1), lambda i: (i, 0)),
                pl.BlockSpec((1, D), lambda i: (0, 0)),
                pl.BlockSpec((D, D), lambda i: (0, 0))],
      out_specs=pl.BlockSpec((BR, D), lambda i: (i, 0)),
      out_shape=jax.ShapeDtypeStruct((N, D), jnp.float32),
  )(parts, tp, dinv, b2d, wn)


def _tc_final(parts, tp, dinv, b2d):
  def body(p_ref, t_ref, dv_ref, b_ref, o_ref):
    s = p_ref[0] + p_ref[1] + t_ref[...]
    o_ref[...] = dv_ref[...] * s + b_ref[...]

  return pl.pallas_call(
      body,
      grid=(N // BR,),
      in_specs=[pl.BlockSpec((2, BR, D), lambda i: (0, i, 0)),
                pl.BlockSpec((BR, D), lambda i: (i, 0)),
                pl.BlockSpec((BR, 1), lambda i: (i, 0)),
                pl.BlockSpec((1, D), lambda i: (0, 0))],
      out_specs=pl.BlockSpec((BR, D), lambda i: (i, 0)),
      out_shape=jax.ShapeDtypeStruct((N, D), jnp.float32),
  )(parts, tp, dinv, b2d)


def kernel(x, edge_index, batch, W1, b1, W2, b2, W3, b3):
  src = edge_index[0].astype(jnp.int32)
  dst = edge_index[1].astype(jnp.int32)
  src_f = jnp.concatenate(
      [src, jnp.zeros((EPAD - E,), jnp.int32)]).reshape(NW * CPW, CHUNK)
  dst_f = jnp.concatenate(
      [dst, jnp.full((EPAD - E,), JUNK, jnp.int32)]).reshape(NW * CPW, CHUNK)
  ei3 = jnp.stack([src_f, dst_f], axis=1)
  ones_rows = jnp.ones((CHUNK, D), jnp.float32)
  zD = jnp.zeros((ZROWS, D), jnp.float32)
  b1r, b2r, b3r = b1.reshape(1, D), b2.reshape(1, D), b3.reshape(1, D)

  t1 = _tc_mm(x, W1)
  degp = _sc_degree(dst_f, ones_rows, zD).reshape(2, NPAD, D)
  dinv, t1p = _tc_scale(degp, t1)
  p1 = _sc_agg(t1p, ei3, zD).reshape(2, NPAD, D)
  t2p = _tc_layer(p1, t1p, dinv, b1r, W2)
  p2 = _sc_agg(t2p, ei3, zD).reshape(2, NPAD, D)
  t3p = _tc_layer(p2, t2p, dinv, b2r, W3)
  p3 = _sc_agg(t3p, ei3, zD).reshape(2, NPAD, D)
  return _tc_final(p3, t3p, dinv, b3r)


# speedup vs baseline: 9.1801x; 1.0393x over previous
import functools

import jax
import jax.numpy as jnp
from jax import lax
from jax.experimental import pallas as pl
from jax.experimental.pallas import tpu as pltpu
from jax.experimental.pallas import tpu_sc as plsc

N = 10000
E = 320000
D = 128
NW = 32
CHUNK = 128
CPW = 80
CPW0 = 124
CPW1 = 36
EPAD = NW * CPW * CHUNK
JUNK = N
NPAD = 10240
RPT = NPAD // 16
ZROWS = 32
BR = 1000

_mesh = plsc.VectorSubcoreMesh(
    core_axis_name="c", subcore_axis_name="s", num_cores=2, num_subcores=16)


KBAT = 16


def _zero_acc(acc, z_hbm, z_v, base, zsem):
  pltpu.sync_copy(z_hbm, z_v)
  nz = RPT // ZROWS
  for r in range(nz):
    pltpu.async_copy(z_v, acc.at[pl.ds(base + r * ZROWS, ZROWS)], zsem)
  for r in range(nz):
    pltpu.make_async_copy(z_v, acc.at[pl.ds(base + r * ZROWS, ZROWS)],
                          zsem).wait()


def _sc_degree(dst2, ones_rows, zrows):
  out_t = jax.ShapeDtypeStruct((2 * NPAD, D), jnp.float32)

  @functools.partial(
      pl.kernel, out_type=out_t, mesh=_mesh,
      scratch_types=[
          pltpu.VMEM((CPW, CHUNK), jnp.int32),
          pltpu.VMEM((CHUNK, D), jnp.float32),
          pltpu.VMEM((ZROWS, D), jnp.float32),
          pltpu.VMEM_SHARED((NPAD, D), jnp.float32),
          pltpu.SemaphoreType.DMA,
          pltpu.SemaphoreType.DMA,
      ])
  def k(dst_hbm, ones_hbm, z_hbm, out_hbm, didx, ones_v, z_v, acc, sem, zsem):
    cid = lax.axis_index("c")
    sid = lax.axis_index("s")
    wid = sid * 2 + cid
    pltpu.sync_copy(dst_hbm.at[pl.ds(wid * CPW, CPW)], didx)
    pltpu.sync_copy(ones_hbm, ones_v)
    base = sid * RPT
    _zero_acc(acc, z_hbm, z_v, base, zsem)
    plsc.subcore_barrier()

    @pl.loop(0, CPW, step=KBAT)
    def _(c):
      for b in range(KBAT):
        pltpu.async_copy(ones_v, acc.at[didx.at[c + b]], sem, add=True)
      for b in range(KBAT):
        pltpu.make_async_copy(ones_v, acc.at[didx.at[c + b]], sem).wait()

    plsc.subcore_barrier()
    pltpu.sync_copy(acc.at[pl.ds(base, RPT)],
                    out_hbm.at[pl.ds(cid * NPAD + base, RPT)])

  return k(dst2, ones_rows, zrows)


def _sc_agg(tp, ei3, zrows):
  out_t = jax.ShapeDtypeStruct((2 * NPAD, D), jnp.float32)

  @functools.partial(
      pl.kernel, out_type=out_t, mesh=_mesh,
      scratch_types=[
          [pltpu.VMEM((2, CHUNK), jnp.int32)] * 4,
          [pltpu.VMEM((CHUNK, D), jnp.float32)] * 2,
          pltpu.VMEM((ZROWS, D), jnp.float32),
          pltpu.VMEM_SHARED((NPAD, D), jnp.float32),
          [pltpu.SemaphoreType.DMA] * 4,
          [pltpu.SemaphoreType.DMA] * 2,
          pltpu.SemaphoreType.DMA,
      ])
  def k(tp_hbm, ei_hbm, z_hbm, out_hbm, ibuf, rows, z_v, acc,
        isem, gsem, zsem):
    cid = lax.axis_index("c")
    sid = lax.axis_index("s")
    nw = jnp.where(cid == 0, CPW0, CPW1)
    e0 = jnp.where(cid == 0, sid * CPW0, 16 * CPW0 + sid * CPW1)
    base = sid * RPT
    _zero_acc(acc, z_hbm, z_v, base, zsem)
    plsc.subcore_barrier()

    for b in range(2):
      pltpu.sync_copy(ei_hbm.at[e0 + b], ibuf[b])
      pltpu.async_copy(tp_hbm.at[ibuf[b].at[0]], rows[b], gsem[b])
    for b in range(2, 4):
      pltpu.async_copy(ei_hbm.at[e0 + b], ibuf[b], isem[b])

    def step(c, b4, more_idx):
      b2 = b4 % 2
      nb4 = (b4 + 2) % 4
      pltpu.make_async_copy(tp_hbm.at[ibuf[b2].at[0]], rows[b2],
                            gsem[b2]).wait()
      pltpu.sync_copy(rows[b2], acc.at[ibuf[b4].at[1]], add=True)
      if more_idx:
        pltpu.async_copy(ei_hbm.at[e0 + c + 4], ibuf[b4], isem[b4])
      pltpu.make_async_copy(ei_hbm.at[e0], ibuf[nb4], isem[nb4]).wait()
      pltpu.async_copy(tp_hbm.at[ibuf[nb4].at[0]], rows[b2], gsem[b2])

    @pl.loop(0, nw - 4, step=4)
    def _(c):
      for b4 in range(4):
        step(c + b4, b4, more_idx=True)

    for b4 in range(4):
      b2 = b4 % 2
      pltpu.make_async_copy(tp_hbm.at[ibuf[b2].at[0]], rows[b2],
                            gsem[b2]).wait()
      pltpu.sync_copy(rows[b2], acc.at[ibuf[b4].at[1]], add=True)
      if b4 < 2:
        nb4 = (b4 + 2) % 4
        pltpu.make_async_copy(ei_hbm.at[e0], ibuf[nb4], isem[nb4]).wait()
        pltpu.async_copy(tp_hbm.at[ibuf[nb4].at[0]], rows[b2], gsem[b2])

    plsc.subcore_barrier()
    pltpu.sync_copy(acc.at[pl.ds(base, RPT)],
                    out_hbm.at[pl.ds(cid * NPAD + base, RPT)])

  return k(tp, ei3, zrows)


def _tc_mm(x, w):
  def body(x_ref, w_ref, o_ref):
    o_ref[...] = jnp.dot(x_ref[...], w_ref[...],
                         preferred_element_type=jnp.float32)

  return pl.pallas_call(
      body,
      grid=(N // BR,),
      in_specs=[pl.BlockSpec((BR, D), lambda i: (i, 0)),
                pl.BlockSpec((D, D), lambda i: (0, 0))],
      out_specs=pl.BlockSpec((BR, D), lambda i: (i, 0)),
      out_shape=jax.ShapeDtypeStruct((N, D), jnp.float32),
  )(x, w)


def _tc_scale(degp, t):
  def body(dg_ref, t_ref, dv_ref, tp_ref):
    dsum = dg_ref[0, :, 0:1] + dg_ref[1, :, 0:1] + 1.0
    dv = lax.rsqrt(dsum)
    dv_ref[...] = dv
    tp_ref[...] = t_ref[...] * dv

  return pl.pallas_call(
      body,
      grid=(N // BR,),
      in_specs=[pl.BlockSpec((2, BR, D), lambda i: (0, i, 0)),
                pl.BlockSpec((BR, D), lambda i: (i, 0))],
      out_specs=[pl.BlockSpec((BR, 1), lambda i: (i, 0)),
                 pl.BlockSpec((BR, D), lambda i: (i, 0))],
      out_shape=[jax.ShapeDtypeStruct((N, 1), jnp.float32),
                 jax.ShapeDtypeStruct((N, D), jnp.float32)],
  )(degp, t)


def _tc_layer(parts, tp, dinv, b2d, wn):
  def body(p_ref, t_ref, dv_ref, b_ref, w_ref, o_ref):
    dv = dv_ref[...]
    s = p_ref[0] + p_ref[1] + t_ref[...]
    h = jnp.maximum(dv * s + b_ref[...], 0.0)
    o_ref[...] = jnp.dot(h, w_ref[...],
                         preferred_element_type=jnp.float32) * dv

  return pl.pallas_call(
      body,
      grid=(N // BR,),
      in_specs=[pl.BlockSpec((2, BR, D), lambda i: (0, i, 0)),
                pl.BlockSpec((BR, D), lambda i: (i, 0)),
                pl.BlockSpec((BR, 1), lambda i: (i, 0)),
                pl.BlockSpec((1, D), lambda i: (0, 0)),
                pl.BlockSpec((D, D), lambda i: (0, 0))],
      out_specs=pl.BlockSpec((BR, D), lambda i: (i, 0)),
      out_shape=jax.ShapeDtypeStruct((N, D), jnp.float32),
  )(parts, tp, dinv, b2d, wn)


def _tc_final(parts, tp, dinv, b2d):
  def body(p_ref, t_ref, dv_ref, b_ref, o_ref):
    s = p_ref[0] + p_ref[1] + t_ref[...]
    o_ref[...] = dv_ref[...] * s + b_ref[...]

  return pl.pallas_call(
      body,
      grid=(N // BR,),
      in_specs=[pl.BlockSpec((2, BR, D), lambda i: (0, i, 0)),
                pl.BlockSpec((BR, D), lambda i: (i, 0)),
                pl.BlockSpec((BR, 1), lambda i: (i, 0)),
                pl.BlockSpec((1, D), lambda i: (0, 0))],
      out_specs=pl.BlockSpec((BR, D), lambda i: (i, 0)),
      out_shape=jax.ShapeDtypeStruct((N, D), jnp.float32),
  )(parts, tp, dinv, b2d)


def kernel(x, edge_index, batch, W1, b1, W2, b2, W3, b3):
  src = edge_index[0].astype(jnp.int32)
  dst = edge_index[1].astype(jnp.int32)
  src_f = jnp.concatenate(
      [src, jnp.zeros((EPAD - E,), jnp.int32)]).reshape(NW * CPW, CHUNK)
  dst_f = jnp.concatenate(
      [dst, jnp.full((EPAD - E,), JUNK, jnp.int32)]).reshape(NW * CPW, CHUNK)
  ei3 = jnp.stack([src_f, dst_f], axis=1)
  ones_rows = jnp.ones((CHUNK, D), jnp.float32)
  zD = jnp.zeros((ZROWS, D), jnp.float32)
  b1r, b2r, b3r = b1.reshape(1, D), b2.reshape(1, D), b3.reshape(1, D)

  t1 = _tc_mm(x, W1)
  degp = _sc_degree(dst_f, ones_rows, zD).reshape(2, NPAD, D)
  dinv, t1p = _tc_scale(degp, t1)
  p1 = _sc_agg(t1p, ei3, zD).reshape(2, NPAD, D)
  t2p = _tc_layer(p1, t1p, dinv, b1r, W2)
  p2 = _sc_agg(t2p, ei3, zD).reshape(2, NPAD, D)
  t3p = _tc_layer(p2, t2p, dinv, b2r, W3)
  p3 = _sc_agg(t3p, ei3, zD).reshape(2, NPAD, D)
  return _tc_final(p3, t3p, dinv, b3r)
